# baseline (speedup 1.0000x reference)
import functools

import jax
import jax.numpy as jnp
from jax import lax
from jax.experimental import pallas as pl
from jax.experimental.pallas import tpu as pltpu
from jax.experimental.pallas import tpu_sc as plsc

EMB = 128
N_NODES = 100000
N_UNARY = 200000
N_BINARY = 200000
N_OUT = N_UNARY + 2 * N_BINARY

NC = 2
NS = 16
NW = NC * NS

GU = 56
GB = 104
U_PAD = GU * NW * 128
B_PAD = GB * NW * 128


def _unary_body(x_ref, w1_ref, b1_ref, w2_ref, b2_ref, o_ref):
    x = x_ref[...]
    h = jnp.maximum(
        jnp.dot(x, w1_ref[...], preferred_element_type=jnp.float32) + b1_ref[...],
        0.0,
    )
    o_ref[...] = x + jnp.dot(h, w2_ref[...], preferred_element_type=jnp.float32) + b2_ref[...]


def _binary_body(f_ref, x_ref, w1_ref, b1_ref, w2_ref, b2_ref, o_ref):
    del f_ref
    x = x_ref[...].reshape(-1, 2 * EMB)
    h = jnp.maximum(
        jnp.dot(x, w1_ref[...], preferred_element_type=jnp.float32) + b1_ref[...],
        0.0,
    )
    y = x + jnp.dot(h, w2_ref[...], preferred_element_type=jnp.float32) + b2_ref[...]
    o_ref[...] = y.reshape(-1, EMB)


def _unary_precompute(x, w1, b1, w2, b2):
    blk = 2000
    return pl.pallas_call(
        _unary_body,
        grid=(N_NODES // blk,),
        in_specs=[
            pl.BlockSpec((blk, EMB), lambda i: (i, 0)),
            pl.BlockSpec((EMB, EMB), lambda i: (0, 0)),
            pl.BlockSpec((1, EMB), lambda i: (0, 0)),
            pl.BlockSpec((EMB, EMB), lambda i: (0, 0)),
            pl.BlockSpec((1, EMB), lambda i: (0, 0)),
        ],
        out_specs=pl.BlockSpec((blk, EMB), lambda i: (i, 0)),
        out_shape=jax.ShapeDtypeStruct((N_NODES, EMB), jnp.float32),
    )(x, w1, b1.reshape(1, EMB), w2, b2.reshape(1, EMB))


def _binary_mlp_into(f, eb, w1, b1, w2, b2):
    blk = 1600
    grid = 2 * N_BINARY // blk
    off = N_UNARY // blk
    return pl.pallas_call(
        _binary_body,
        grid=(grid,),
        in_specs=[
            pl.BlockSpec(memory_space=pl.ANY),
            pl.BlockSpec((blk, EMB), lambda i: (i, 0)),
            pl.BlockSpec((2 * EMB, 2 * EMB), lambda i: (0, 0)),
            pl.BlockSpec((1, 2 * EMB), lambda i: (0, 0)),
            pl.BlockSpec((2 * EMB, 2 * EMB), lambda i: (0, 0)),
            pl.BlockSpec((1, 2 * EMB), lambda i: (0, 0)),
        ],
        out_specs=pl.BlockSpec((blk, EMB), lambda i, off=off: (i + off, 0)),
        out_shape=jax.ShapeDtypeStruct((N_OUT, EMB), jnp.float32),
        input_output_aliases={0: 0},
    )(f, eb, w1, b1.reshape(1, 2 * EMB), w2, b2.reshape(1, 2 * EMB))


NB = 4


def _sc_gather(x, u, idx_u, idx_b):
    mesh = plsc.VectorSubcoreMesh(core_axis_name="c", subcore_axis_name="s")

    @functools.partial(
        pl.kernel,
        mesh=mesh,
        out_type=[
            jax.ShapeDtypeStruct((N_OUT, EMB), jnp.float32),
            jax.ShapeDtypeStruct((B_PAD, EMB), jnp.float32),
        ],
        scratch_types=(
            [pltpu.VMEM((GB, 128), jnp.int32)]
            + [pltpu.VMEM((128, EMB), jnp.float32) for _ in range(NB)]
            + [pltpu.SemaphoreType.DMA for _ in range(2 * NB)]
        ),
    )
    def k(x_hbm, u_hbm, idxu_hbm, idxb_hbm, f_hbm, eb_hbm, idx_all,
          rv0, rv1, rv2, rv3, sg0, sg1, sg2, sg3, so0, so1, so2, so3):
        wid = lax.axis_index("s") * NC + lax.axis_index("c")
        rvs = (rv0, rv1, rv2, rv3)
        sgs = (sg0, sg1, sg2, sg3)
        sos = (so0, so1, so2, so3)

        def region(table, idx_hbm, out_hbm, n_units):
            pltpu.sync_copy(
                idx_hbm.at[pl.ds(wid * n_units, n_units)],
                idx_all.at[pl.ds(0, n_units)],
            )
            base = wid * n_units * 128

            def drain_gather_and_fire_out(g, b2):
                pltpu.make_async_copy(
                    out_hbm.at[pl.ds(0, 128)], rvs[b2], sgs[b2]
                ).wait()
                pltpu.async_copy(
                    rvs[b2], out_hbm.at[pl.ds(base + g * 128, 128)], sos[b2]
                )

            def body(kk, carry):
                for b in range(NB):
                    g = kk * NB + b

                    @pl.when(g >= NB)
                    def _():
                        pltpu.make_async_copy(
                            out_hbm.at[pl.ds(0, 128)], rvs[b], sos[b]
                        ).wait()

                    pltpu.async_copy(table.at[idx_all.at[g]], rvs[b], sgs[b])

                    @pl.when(g >= 2)
                    def _():
                        drain_gather_and_fire_out(g - 2, (b + NB - 2) % NB)
                return carry

            lax.fori_loop(0, n_units // NB, body, 0)
            for i in range(2):
                drain_gather_and_fire_out(n_units - 2 + i, (NB - 2 + i) % NB)
            for b in range(NB):
                pltpu.make_async_copy(
                    out_hbm.at[pl.ds(0, 128)], rvs[b], sos[b]
                ).wait()

        region(u_hbm, idxu_hbm, f_hbm, GU)
        region(x_hbm, idxb_hbm, eb_hbm, GB)

    return k(x, u, idx_u, idx_b)


def kernel(node_embeddings, rel_unary_indices, rel_binary_indices,
           u_W1, u_b1, u_W2, u_b2, b_W1, b_b1, b_W2, b_b2):
    x = node_embeddings
    u = _unary_precompute(x, u_W1, u_b1, u_W2, u_b2)

    idx_u = jnp.concatenate(
        [rel_unary_indices.astype(jnp.int32),
         jnp.zeros((U_PAD - N_UNARY,), jnp.int32)]
    ).reshape(-1, 128)
    idx_b = jnp.concatenate(
        [rel_binary_indices.astype(jnp.int32),
         jnp.zeros((B_PAD - 2 * N_BINARY,), jnp.int32)]
    ).reshape(-1, 128)

    f, eb = _sc_gather(x, u, idx_u, idx_b)
    out = _binary_mlp_into(f, eb, b_W1, b_b1, b_W2, b_b2)

    output_indices = jnp.concatenate([rel_unary_indices, rel_binary_indices])
    return out, output_indices


# speedup vs baseline: 2.3204x; 2.3204x over previous
import functools

import jax
import jax.numpy as jnp
from jax import lax
from jax.experimental import pallas as pl
from jax.experimental.pallas import tpu as pltpu
from jax.experimental.pallas import tpu_sc as plsc

EMB = 128
N_NODES = 100000
N_UNARY = 200000
N_BINARY = 200000
N_OUT = N_UNARY + 2 * N_BINARY

NC = 2
NS = 16
NW = NC * NS

CH = 384

KU = 17
KB = 33
U_PAD = KU * NW * CH
B_PAD = KB * NW * CH


def _unary_body(x_ref, w1_ref, b1_ref, w2_ref, b2_ref, o_ref):
    x = x_ref[...]
    h = jnp.maximum(
        jnp.dot(x, w1_ref[...], preferred_element_type=jnp.float32) + b1_ref[...],
        0.0,
    )
    o_ref[...] = x + jnp.dot(h, w2_ref[...], preferred_element_type=jnp.float32) + b2_ref[...]


def _binary_body(f_ref, x_ref, w1_ref, b1_ref, w2_ref, b2_ref, o_ref):
    del f_ref
    x = x_ref[...].reshape(-1, 2 * EMB)
    h = jnp.maximum(
        jnp.dot(x, w1_ref[...], preferred_element_type=jnp.float32) + b1_ref[...],
        0.0,
    )
    y = x + jnp.dot(h, w2_ref[...], preferred_element_type=jnp.float32) + b2_ref[...]
    o_ref[...] = y.reshape(-1, EMB)


def _unary_precompute(x, w1, b1, w2, b2):
    blk = 2000
    return pl.pallas_call(
        _unary_body,
        grid=(N_NODES // blk,),
        in_specs=[
            pl.BlockSpec((blk, EMB), lambda i: (i, 0)),
            pl.BlockSpec((EMB, EMB), lambda i: (0, 0)),
            pl.BlockSpec((1, EMB), lambda i: (0, 0)),
            pl.BlockSpec((EMB, EMB), lambda i: (0, 0)),
            pl.BlockSpec((1, EMB), lambda i: (0, 0)),
        ],
        out_specs=pl.BlockSpec((blk, EMB), lambda i: (i, 0)),
        out_shape=jax.ShapeDtypeStruct((N_NODES, EMB), jnp.float32),
    )(x, w1, b1.reshape(1, EMB), w2, b2.reshape(1, EMB))


def _binary_mlp_into(f, eb, w1, b1, w2, b2):
    blk = 1600
    grid = 2 * N_BINARY // blk
    off = N_UNARY // blk
    return pl.pallas_call(
        _binary_body,
        grid=(grid,),
        in_specs=[
            pl.BlockSpec(memory_space=pl.ANY),
            pl.BlockSpec((blk, EMB), lambda i: (i, 0)),
            pl.BlockSpec((2 * EMB, 2 * EMB), lambda i: (0, 0)),
            pl.BlockSpec((1, 2 * EMB), lambda i: (0, 0)),
            pl.BlockSpec((2 * EMB, 2 * EMB), lambda i: (0, 0)),
            pl.BlockSpec((1, 2 * EMB), lambda i: (0, 0)),
        ],
        out_specs=pl.BlockSpec((blk, EMB), lambda i, off=off: (i + off, 0)),
        out_shape=jax.ShapeDtypeStruct((N_OUT, EMB), jnp.float32),
        input_output_aliases={0: 0},
    )(f, eb, w1, b1.reshape(1, 2 * EMB), w2, b2.reshape(1, 2 * EMB))


def _sc_gather(x, u, idx_u, idx_b):
    mesh = plsc.VectorSubcoreMesh(core_axis_name="c", subcore_axis_name="s")

    @functools.partial(
        pl.kernel,
        mesh=mesh,
        out_type=[
            jax.ShapeDtypeStruct((N_OUT, EMB), jnp.float32),
            jax.ShapeDtypeStruct((B_PAD, EMB), jnp.float32),
        ],
        scratch_types=[
            pltpu.VMEM((CH,), jnp.int32),
            pltpu.VMEM((CH,), jnp.int32),
            pltpu.VMEM((CH, EMB), jnp.float32),
            pltpu.VMEM((CH, EMB), jnp.float32),
            pltpu.SemaphoreType.DMA,
            pltpu.SemaphoreType.DMA,
            pltpu.SemaphoreType.DMA,
            pltpu.SemaphoreType.DMA,
            pltpu.SemaphoreType.DMA,
            pltpu.SemaphoreType.DMA,
        ],
    )
    def k(x_hbm, u_hbm, idxu_hbm, idxb_hbm, f_hbm, eb_hbm,
          idx_v0, idx_v1, rows_v0, rows_v1,
          sem_i0, sem_i1, sem_g0, sem_g1, sem_o0, sem_o1):
        wid = lax.axis_index("s") * NC + lax.axis_index("c")
        bufs = (
            (idx_v0, rows_v0, sem_i0, sem_g0, sem_o0, idx_v1, sem_i1),
            (idx_v1, rows_v1, sem_i1, sem_g1, sem_o1, idx_v0, sem_i0),
        )

        def region(table, idx_hbm, out_hbm, k_per_worker):
            def slot(t, b):
                idxv, rowsv, semi, semg, semo, idxv_n, semi_n = bufs[b]
                tt = jnp.int32(t)

                @pl.when(tt >= 2)
                def _():
                    pltpu.make_async_copy(
                        out_hbm.at[pl.ds(0, CH)], rowsv, semo
                    ).wait()

                pltpu.make_async_copy(
                    idx_hbm.at[pl.ds(0, CH)], idxv, semi
                ).wait()
                gcop = pltpu.async_copy(table.at[idxv], rowsv, semg)

                @pl.when(tt + 1 < k_per_worker)
                def _():
                    nxt = (wid + NW * (tt + 1)) * CH
                    pltpu.async_copy(
                        idx_hbm.at[pl.ds(nxt, CH)], idxv_n, semi_n
                    )

                gcop.wait()
                pltpu.async_copy(
                    rowsv, out_hbm.at[pl.ds((wid + NW * tt) * CH, CH)], semo
                )

            def body(kk, carry):
                slot(2 * kk, 0)
                slot(2 * kk + 1, 1)
                return carry

            pltpu.async_copy(idx_hbm.at[pl.ds(wid * CH, CH)], idx_v0, sem_i0)
            lax.fori_loop(0, k_per_worker // 2, body, 0)
            if k_per_worker % 2:
                slot(k_per_worker - 1, 0)
            pltpu.make_async_copy(out_hbm.at[pl.ds(0, CH)], rows_v0, sem_o0).wait()
            pltpu.make_async_copy(out_hbm.at[pl.ds(0, CH)], rows_v1, sem_o1).wait()

        region(u_hbm, idxu_hbm, f_hbm, KU)
        region(x_hbm, idxb_hbm, eb_hbm, KB)

    return k(x, u, idx_u, idx_b)


def _pad_idx(idx, n_pad):
    return jnp.concatenate(
        [idx.astype(jnp.int32), jnp.zeros((n_pad - idx.shape[0],), jnp.int32)]
    )


def kernel(node_embeddings, rel_unary_indices, rel_binary_indices,
           u_W1, u_b1, u_W2, u_b2, b_W1, b_b1, b_W2, b_b2):
    x = node_embeddings
    u = _unary_precompute(x, u_W1, u_b1, u_W2, u_b2)

    idx_u = _pad_idx(rel_unary_indices, U_PAD)
    idx_b = _pad_idx(rel_binary_indices, B_PAD)

    f, eb = _sc_gather(x, u, idx_u, idx_b)
    out = _binary_mlp_into(f, eb, b_W1, b_b1, b_W2, b_b2)

    output_indices = jnp.concatenate([rel_unary_indices, rel_binary_indices])
    return out, output_indices


# speedup vs baseline: 4.0986x; 1.7663x over previous
import functools

import jax
import jax.numpy as jnp
from jax import lax
from jax.experimental import pallas as pl
from jax.experimental.pallas import tpu as pltpu
from jax.experimental.pallas import tpu_sc as plsc

EMB = 128
N_NODES = 100000
N_UNARY = 200000
N_BINARY = 200000
N_OUT = N_UNARY + 2 * N_BINARY

NC = 2
NS = 16
NW = NC * NS

CH = 448

KU = 14
KB = 28
U_PAD = KU * NW * CH
B_PAD = KB * NW * CH


def _unary_body(x_ref, w1_ref, b1_ref, w2_ref, b2_ref, o_ref):
    x = x_ref[...]
    h = jnp.maximum(
        jnp.dot(x, w1_ref[...], preferred_element_type=jnp.float32) + b1_ref[...],
        0.0,
    )
    o_ref[...] = x + jnp.dot(h, w2_ref[...], preferred_element_type=jnp.float32) + b2_ref[...]


def _binary_body(f_ref, x_ref, w1_ref, b1_ref, w2_ref, b2_ref, o_ref):
    del f_ref
    x = x_ref[...].reshape(-1, 2 * EMB)
    h = jnp.maximum(
        jnp.dot(x, w1_ref[...], preferred_element_type=jnp.float32) + b1_ref[...],
        0.0,
    )
    y = x + jnp.dot(h, w2_ref[...], preferred_element_type=jnp.float32) + b2_ref[...]
    o_ref[...] = y.reshape(-1, EMB)


def _unary_precompute(x, w1, b1, w2, b2):
    blk = 2000
    return pl.pallas_call(
        _unary_body,
        grid=(N_NODES // blk,),
        in_specs=[
            pl.BlockSpec((blk, EMB), lambda i: (i, 0)),
            pl.BlockSpec((EMB, EMB), lambda i: (0, 0)),
            pl.BlockSpec((1, EMB), lambda i: (0, 0)),
            pl.BlockSpec((EMB, EMB), lambda i: (0, 0)),
            pl.BlockSpec((1, EMB), lambda i: (0, 0)),
        ],
        out_specs=pl.BlockSpec((blk, EMB), lambda i: (i, 0)),
        out_shape=jax.ShapeDtypeStruct((N_NODES, EMB), jnp.float32),
    )(x, w1, b1.reshape(1, EMB), w2, b2.reshape(1, EMB))


def _binary_mlp_into(f, eb, w1, b1, w2, b2):
    blk = 1600
    grid = 2 * N_BINARY // blk
    off = N_UNARY // blk
    return pl.pallas_call(
        _binary_body,
        grid=(grid,),
        in_specs=[
            pl.BlockSpec(memory_space=pl.ANY),
            pl.BlockSpec((blk, EMB), lambda i: (i, 0)),
            pl.BlockSpec((2 * EMB, 2 * EMB), lambda i: (0, 0)),
            pl.BlockSpec((1, 2 * EMB), lambda i: (0, 0)),
            pl.BlockSpec((2 * EMB, 2 * EMB), lambda i: (0, 0)),
            pl.BlockSpec((1, 2 * EMB), lambda i: (0, 0)),
        ],
        out_specs=pl.BlockSpec((blk, EMB), lambda i, off=off: (i + off, 0)),
        out_shape=jax.ShapeDtypeStruct((N_OUT, EMB), jnp.float32),
        input_output_aliases={0: 0},
    )(f, eb, w1, b1.reshape(1, 2 * EMB), w2, b2.reshape(1, 2 * EMB))


def _sc_gather(x, u, idx_u, idx_b):
    mesh = plsc.VectorSubcoreMesh(core_axis_name="c", subcore_axis_name="s")

    @functools.partial(
        pl.kernel,
        mesh=mesh,
        out_type=[
            jax.ShapeDtypeStruct((N_OUT, EMB), jnp.float32),
            jax.ShapeDtypeStruct((B_PAD, EMB), jnp.float32),
        ],
        scratch_types=[
            pltpu.VMEM((CH,), jnp.int32),
            pltpu.VMEM((CH,), jnp.int32),
            pltpu.VMEM((CH, EMB), jnp.float32),
            pltpu.VMEM((CH, EMB), jnp.float32),
            pltpu.SemaphoreType.DMA,
            pltpu.SemaphoreType.DMA,
            pltpu.SemaphoreType.DMA,
            pltpu.SemaphoreType.DMA,
            pltpu.SemaphoreType.DMA,
            pltpu.SemaphoreType.DMA,
        ],
    )
    def k(x_hbm, u_hbm, idxu_hbm, idxb_hbm, f_hbm, eb_hbm,
          idx_v0, idx_v1, rows_v0, rows_v1,
          sem_i0, sem_i1, sem_g0, sem_g1, sem_o0, sem_o1):
        wid = lax.axis_index("s") * NC + lax.axis_index("c")
        bufs = (
            (idx_v0, rows_v0, sem_i0, sem_g0, sem_o0, idx_v1, sem_i1),
            (idx_v1, rows_v1, sem_i1, sem_g1, sem_o1, idx_v0, sem_i0),
        )

        def region(table, idx_hbm, out_hbm, k_per_worker):
            def slot(t, b):
                idxv, rowsv, semi, semg, semo, idxv_n, semi_n = bufs[b]
                tt = jnp.int32(t)

                @pl.when(tt >= 2)
                def _():
                    pltpu.make_async_copy(
                        out_hbm.at[pl.ds(0, CH)], rowsv, semo
                    ).wait()

                pltpu.make_async_copy(
                    idx_hbm.at[pl.ds(0, CH)], idxv, semi
                ).wait()
                gcop = pltpu.async_copy(table.at[idxv], rowsv, semg)

                @pl.when(tt + 1 < k_per_worker)
                def _():
                    nxt = (wid + NW * (tt + 1)) * CH
                    pltpu.async_copy(
                        idx_hbm.at[pl.ds(nxt, CH)], idxv_n, semi_n
                    )

                gcop.wait()
                pltpu.async_copy(
                    rowsv, out_hbm.at[pl.ds((wid + NW * tt) * CH, CH)], semo
                )

            def body(kk, carry):
                slot(2 * kk, 0)
                slot(2 * kk + 1, 1)
                return carry

            pltpu.async_copy(idx_hbm.at[pl.ds(wid * CH, CH)], idx_v0, sem_i0)
            lax.fori_loop(0, k_per_worker // 2, body, 0)
            if k_per_worker % 2:
                slot(k_per_worker - 1, 0)
            pltpu.make_async_copy(out_hbm.at[pl.ds(0, CH)], rows_v0, sem_o0).wait()
            pltpu.make_async_copy(out_hbm.at[pl.ds(0, CH)], rows_v1, sem_o1).wait()

        region(u_hbm, idxu_hbm, f_hbm, KU)
        region(x_hbm, idxb_hbm, eb_hbm, KB)

    return k(x, u, idx_u, idx_b)


def _pad_idx(idx, n_pad):
    return jnp.concatenate(
        [idx.astype(jnp.int32), jnp.zeros((n_pad - idx.shape[0],), jnp.int32)]
    )


def kernel(node_embeddings, rel_unary_indices, rel_binary_indices,
           u_W1, u_b1, u_W2, u_b2, b_W1, b_b1, b_W2, b_b2):
    x = node_embeddings
    u = _unary_precompute(x, u_W1, u_b1, u_W2, u_b2)

    idx_u = _pad_idx(rel_unary_indices, U_PAD)
    idx_b = _pad_idx(rel_binary_indices, B_PAD)

    f, eb = _sc_gather(x, u, idx_u, idx_b)
    out = _binary_mlp_into(f, eb, b_W1, b_b1, b_W2, b_b2)

    output_indices = jnp.concatenate([rel_unary_indices, rel_binary_indices])
    return out, output_indices


# speedup vs baseline: 4.2793x; 1.0441x over previous
import functools

import jax
import jax.numpy as jnp
from jax import lax
from jax.experimental import pallas as pl
from jax.experimental.pallas import tpu as pltpu
from jax.experimental.pallas import tpu_sc as plsc

EMB = 128
N_NODES = 100000
N_UNARY = 200000
N_BINARY = 200000
N_OUT = N_UNARY + 2 * N_BINARY

NC = 2
NS = 16
NW = NC * NS

CH = 448

KU = 14
KB = 28
U_PAD = KU * NW * CH
B_PAD = KB * NW * CH


def _unary_body(x_ref, w1_ref, b1_ref, w2_ref, b2_ref, o_ref):
    x = x_ref[...]
    h = jnp.maximum(
        jnp.dot(x, w1_ref[...], preferred_element_type=jnp.float32) + b1_ref[...],
        0.0,
    )
    o_ref[...] = x + jnp.dot(h, w2_ref[...], preferred_element_type=jnp.float32) + b2_ref[...]


def _binary_body(f_ref, x_ref, w1_ref, b1_ref, w2_ref, b2_ref, o_ref):
    del f_ref
    x = x_ref[...].reshape(-1, 2 * EMB)
    h = jnp.maximum(
        jnp.dot(x, w1_ref[...], preferred_element_type=jnp.float32) + b1_ref[...],
        0.0,
    )
    y = x + jnp.dot(h, w2_ref[...], preferred_element_type=jnp.float32) + b2_ref[...]
    o_ref[...] = y.reshape(-1, EMB)


def _unary_precompute(x, w1, b1, w2, b2):
    blk = 2000
    return pl.pallas_call(
        _unary_body,
        grid=(N_NODES // blk,),
        in_specs=[
            pl.BlockSpec((blk, EMB), lambda i: (i, 0)),
            pl.BlockSpec((EMB, EMB), lambda i: (0, 0)),
            pl.BlockSpec((1, EMB), lambda i: (0, 0)),
            pl.BlockSpec((EMB, EMB), lambda i: (0, 0)),
            pl.BlockSpec((1, EMB), lambda i: (0, 0)),
        ],
        out_specs=pl.BlockSpec((blk, EMB), lambda i: (i, 0)),
        out_shape=jax.ShapeDtypeStruct((N_NODES, EMB), jnp.float32),
    )(x, w1, b1.reshape(1, EMB), w2, b2.reshape(1, EMB))


def _binary_mlp_into(f, eb, w1, b1, w2, b2):
    blk = 2000
    grid = 2 * N_BINARY // blk
    off = N_UNARY // blk
    return pl.pallas_call(
        _binary_body,
        grid=(grid,),
        in_specs=[
            pl.BlockSpec(memory_space=pl.ANY),
            pl.BlockSpec((blk, EMB), lambda i: (i, 0)),
            pl.BlockSpec((2 * EMB, 2 * EMB), lambda i: (0, 0)),
            pl.BlockSpec((1, 2 * EMB), lambda i: (0, 0)),
            pl.BlockSpec((2 * EMB, 2 * EMB), lambda i: (0, 0)),
            pl.BlockSpec((1, 2 * EMB), lambda i: (0, 0)),
        ],
        out_specs=pl.BlockSpec((blk, EMB), lambda i, off=off: (i + off, 0)),
        out_shape=jax.ShapeDtypeStruct((N_OUT, EMB), jnp.float32),
        input_output_aliases={0: 0},
    )(f, eb, w1, b1.reshape(1, 2 * EMB), w2, b2.reshape(1, 2 * EMB))


def _sc_gather(x, u, idx_u, idx_b):
    mesh = plsc.VectorSubcoreMesh(core_axis_name="c", subcore_axis_name="s")

    @functools.partial(
        pl.kernel,
        mesh=mesh,
        out_type=[
            jax.ShapeDtypeStruct((N_OUT, EMB), jnp.float32),
            jax.ShapeDtypeStruct((B_PAD, EMB), jnp.float32),
        ],
        scratch_types=[
            pltpu.VMEM((CH,), jnp.int32),
            pltpu.VMEM((CH,), jnp.int32),
            pltpu.VMEM((CH, EMB), jnp.float32),
            pltpu.VMEM((CH, EMB), jnp.float32),
            pltpu.SemaphoreType.DMA,
            pltpu.SemaphoreType.DMA,
            pltpu.SemaphoreType.DMA,
            pltpu.SemaphoreType.DMA,
            pltpu.SemaphoreType.DMA,
            pltpu.SemaphoreType.DMA,
        ],
    )
    def k(x_hbm, u_hbm, idxu_hbm, idxb_hbm, f_hbm, eb_hbm,
          idx_v0, idx_v1, rows_v0, rows_v1,
          sem_i0, sem_i1, sem_g0, sem_g1, sem_o0, sem_o1):
        wid = lax.axis_index("s") * NC + lax.axis_index("c")
        bufs = (
            (idx_v0, rows_v0, sem_i0, sem_g0, sem_o0, idx_v1, sem_i1),
            (idx_v1, rows_v1, sem_i1, sem_g1, sem_o1, idx_v0, sem_i0),
        )

        def region(table, idx_hbm, out_hbm, k_per_worker):
            def slot(t, b):
                idxv, rowsv, semi, semg, semo, idxv_n, semi_n = bufs[b]
                tt = jnp.int32(t)

                @pl.when(tt >= 2)
                def _():
                    pltpu.make_async_copy(
                        out_hbm.at[pl.ds(0, CH)], rowsv, semo
                    ).wait()

                pltpu.make_async_copy(
                    idx_hbm.at[pl.ds(0, CH)], idxv, semi
                ).wait()
                gcop = pltpu.async_copy(table.at[idxv], rowsv, semg)

                @pl.when(tt + 1 < k_per_worker)
                def _():
                    nxt = (wid + NW * (tt + 1)) * CH
                    pltpu.async_copy(
                        idx_hbm.at[pl.ds(nxt, CH)], idxv_n, semi_n
                    )

                gcop.wait()
                pltpu.async_copy(
                    rowsv, out_hbm.at[pl.ds((wid + NW * tt) * CH, CH)], semo
                )

            def body(kk, carry):
                slot(2 * kk, 0)
                slot(2 * kk + 1, 1)
                return carry

            pltpu.async_copy(idx_hbm.at[pl.ds(wid * CH, CH)], idx_v0, sem_i0)
            lax.fori_loop(0, k_per_worker // 2, body, 0)
            if k_per_worker % 2:
                slot(k_per_worker - 1, 0)
            pltpu.make_async_copy(out_hbm.at[pl.ds(0, CH)], rows_v0, sem_o0).wait()
            pltpu.make_async_copy(out_hbm.at[pl.ds(0, CH)], rows_v1, sem_o1).wait()

        region(u_hbm, idxu_hbm, f_hbm, KU)
        region(x_hbm, idxb_hbm, eb_hbm, KB)

    return k(x, u, idx_u, idx_b)


def _pad_idx(idx, n_pad):
    return jnp.concatenate(
        [idx.astype(jnp.int32), jnp.zeros((n_pad - idx.shape[0],), jnp.int32)]
    )


def kernel(node_embeddings, rel_unary_indices, rel_binary_indices,
           u_W1, u_b1, u_W2, u_b2, b_W1, b_b1, b_W2, b_b2):
    x = node_embeddings
    u = _unary_precompute(x, u_W1, u_b1, u_W2, u_b2)

    idx_u = _pad_idx(rel_unary_indices, U_PAD)
    idx_b = _pad_idx(rel_binary_indices, B_PAD)

    f, eb = _sc_gather(x, u, idx_u, idx_b)
    out = _binary_mlp_into(f, eb, b_W1, b_b1, b_W2, b_b2)

    output_indices = jnp.concatenate([rel_unary_indices, rel_binary_indices])
    return out, output_indices
